# baseline (speedup 1.0000x reference)
import functools

import jax
import jax.numpy as jnp
from jax import lax
from jax.experimental import pallas as pl
from jax.experimental.pallas import tpu as pltpu
from jax.experimental.pallas import tpu_sc as plsc

B = 16384
NUM_ACTIONS = 6
NC, NS, L = 2, 16, 16
ROWS_PER_SUB = B // (NC * NS)
CHUNKS = ROWS_PER_SUB // L

_LN2 = 0.6931471805599453
_SQRT2 = 1.4142135623730951


def _log_f32(x):
    xi = plsc.bitcast(x, jnp.int32)
    e = (xi >> 23) - 127
    m = plsc.bitcast((xi & 0x007FFFFF) | 0x3F800000, jnp.float32)
    big = m > _SQRT2
    m = jnp.where(big, m * 0.5, m)
    e = e + jnp.where(big, 1, 0)
    s = (m - 1.0) / (m + 1.0)
    z = s * s
    ln_m = s * (2.0 + z * (2.0 / 3.0 + z * (0.4 + z * (2.0 / 7.0))))
    return e.astype(jnp.float32) * _LN2 + ln_m


def _sc_body(
    q_hbm, a_hbm, r_hbm, out_hbm,
    q_v, a_v, r_v, acc_v, shared, sum_v, cnt, sem_q, sem_a, sem_r,
):
    sid = lax.axis_index("s")
    cid = lax.axis_index("c")
    base = (cid * NS + sid) * ROWS_PER_SUB

    @pl.when(sid == 0)
    def _():
        cnt[0] = 0

    plsc.subcore_barrier()
    cq = pltpu.async_copy(
        q_hbm.at[pl.ds(base * NUM_ACTIONS, ROWS_PER_SUB * NUM_ACTIONS)], q_v, sem_q
    )
    ca = pltpu.async_copy(a_hbm.at[pl.ds(base, ROWS_PER_SUB)], a_v, sem_a)
    cr = pltpu.async_copy(r_hbm.at[pl.ds(base, ROWS_PER_SUB)], r_v, sem_r)
    cq.wait()
    ca.wait()
    cr.wait()

    def _chunk(c, acc):
        rows = c * L + lax.iota(jnp.int32, L)
        acts = a_v[pl.ds(c * L, L)]
        rew = r_v[pl.ds(c * L, L)]
        g = plsc.load_gather(q_v, [rows * NUM_ACTIONS + acts])
        return acc + _log_f32(g) * rew

    acc = lax.fori_loop(0, CHUNKS, _chunk, jnp.zeros((L,), jnp.float32), unroll=4)

    acc_v[...] = acc
    pltpu.sync_copy(acc_v, shared.at[pl.ds((cid * NS + sid) * L, L)])
    plsc.fetch_and_add(cnt.at[0], jnp.int32(1), subcore_id=jnp.int32(0))

    @pl.when(sid == 0)
    def _():
        lax.while_loop(lambda c: c < NS, lambda c: cnt[0], cnt[0])

        pltpu.sync_copy(shared.at[pl.ds(cid * NS * L, NS * L)], sum_v)
        first = tuple(sum_v[pl.ds(i * L, L)] for i in range(NS))

        def _reread(carry):
            prev = carry[1:]
            pltpu.sync_copy(shared.at[pl.ds(cid * NS * L, NS * L)], sum_v)
            rows = tuple(sum_v[pl.ds(i * L, L)] for i in range(NS))
            stable = jnp.bool_(True)
            for p, r in zip(prev, rows):
                stable = jnp.logical_and(
                    stable,
                    jnp.all(plsc.bitcast(p, jnp.int32)
                            == plsc.bitcast(r, jnp.int32)),
                )
            return (stable,) + rows

        final = lax.while_loop(
            lambda c: jnp.logical_not(c[0]), _reread, (jnp.bool_(False),) + first
        )

        tot = final[1]
        for i in range(2, NS + 1):
            tot = tot + final[i]
        half = jnp.sum(tot)
        acc_v[...] = jnp.where(lax.iota(jnp.int32, L) == 0, half, 0.0)
        pltpu.sync_copy(acc_v.at[pl.ds(0, 8)], out_hbm.at[pl.ds(cid * 8, 8)])


_sc_loss = functools.partial(
    pl.kernel,
    mesh=plsc.VectorSubcoreMesh(core_axis_name="c", subcore_axis_name="s"),
    out_type=jax.ShapeDtypeStruct((L,), jnp.float32),
    compiler_params=pltpu.CompilerParams(needs_layout_passes=False),
    scratch_types=[
        pltpu.VMEM((ROWS_PER_SUB * NUM_ACTIONS,), jnp.float32),
        pltpu.VMEM((ROWS_PER_SUB,), jnp.int32),
        pltpu.VMEM((ROWS_PER_SUB,), jnp.float32),
        pltpu.VMEM((L,), jnp.float32),
        pltpu.VMEM_SHARED((NC * NS * L,), jnp.float32),
        pltpu.VMEM((NS * L,), jnp.float32),
        pltpu.SMEM((1,), jnp.int32),
        pltpu.SemaphoreType.DMA,
        pltpu.SemaphoreType.DMA,
        pltpu.SemaphoreType.DMA,
    ],
)(_sc_body)


def _finish_body(p_ref, o_ref):
    o_ref[...] = (jnp.sum(p_ref[...]) * (-1.0 / B)).reshape(1, 1)


_finish = pl.pallas_call(
    _finish_body,
    out_shape=jax.ShapeDtypeStruct((1, 1), jnp.float32),
)


def kernel(q_pred, true_action, discounted_reward):
    out = _sc_loss(
        q_pred.reshape(B * NUM_ACTIONS), true_action.reshape(B), discounted_reward
    )
    return _finish(out)[0, 0]


# speedup vs baseline: 1.0242x; 1.0242x over previous
import functools

import jax
import jax.numpy as jnp
from jax import lax
from jax.experimental import pallas as pl
from jax.experimental.pallas import tpu as pltpu
from jax.experimental.pallas import tpu_sc as plsc

B = 16384
NUM_ACTIONS = 6
NS, L = 16, 16
ROWS_PER_SUB = B // NS
CHUNKS = ROWS_PER_SUB // L

_LN2 = 0.6931471805599453
_SQRT2 = 1.4142135623730951


def _log_f32(x):
    xi = plsc.bitcast(x, jnp.int32)
    e = (xi >> 23) - 127
    m = plsc.bitcast((xi & 0x007FFFFF) | 0x3F800000, jnp.float32)
    big = m > _SQRT2
    m = jnp.where(big, m * 0.5, m)
    e = e + jnp.where(big, 1, 0)
    s = (m - 1.0) / (m + 1.0)
    z = s * s
    ln_m = s * (2.0 + z * (2.0 / 3.0 + z * (0.4 + z * (2.0 / 7.0))))
    return e.astype(jnp.float32) * _LN2 + ln_m


def _sc_body(
    q_hbm, a_hbm, r_hbm, out_hbm,
    q_v, a_v, r_v, acc_v, shared, sum_v, cnt, sem_q, sem_a, sem_r,
):
    sid = lax.axis_index("s")
    base = sid * ROWS_PER_SUB

    @pl.when(sid == 0)
    def _():
        cnt[0] = 0

    plsc.subcore_barrier()
    cq = pltpu.async_copy(
        q_hbm.at[pl.ds(base * NUM_ACTIONS, ROWS_PER_SUB * NUM_ACTIONS)], q_v, sem_q
    )
    ca = pltpu.async_copy(a_hbm.at[pl.ds(base, ROWS_PER_SUB)], a_v, sem_a)
    cr = pltpu.async_copy(r_hbm.at[pl.ds(base, ROWS_PER_SUB)], r_v, sem_r)
    cq.wait()
    ca.wait()
    cr.wait()

    def _chunk(c, acc):
        rows = c * L + lax.iota(jnp.int32, L)
        acts = a_v[pl.ds(c * L, L)]
        rew = r_v[pl.ds(c * L, L)]
        g = plsc.load_gather(q_v, [rows * NUM_ACTIONS + acts])
        return acc + _log_f32(g) * rew

    acc = lax.fori_loop(0, CHUNKS, _chunk, jnp.zeros((L,), jnp.float32), unroll=4)

    acc_v[...] = acc
    pltpu.sync_copy(acc_v, shared.at[pl.ds(sid * L, L)])
    plsc.fetch_and_add(cnt.at[0], jnp.int32(1), subcore_id=jnp.int32(0))

    @pl.when(sid == 0)
    def _():
        lax.while_loop(lambda c: c < NS, lambda c: cnt[0], cnt[0])

        pltpu.sync_copy(shared, sum_v)
        first = tuple(sum_v[pl.ds(i * L, L)] for i in range(NS))

        def _reread(carry):
            prev = carry[1:]
            pltpu.sync_copy(shared, sum_v)
            rows = tuple(sum_v[pl.ds(i * L, L)] for i in range(NS))
            stable = jnp.bool_(True)
            for p, r in zip(prev, rows):
                stable = jnp.logical_and(
                    stable,
                    jnp.all(plsc.bitcast(p, jnp.int32)
                            == plsc.bitcast(r, jnp.int32)),
                )
            return (stable,) + rows

        final = lax.while_loop(
            lambda c: jnp.logical_not(c[0]), _reread, (jnp.bool_(False),) + first
        )

        tot = final[1]
        for i in range(2, NS + 1):
            tot = tot + final[i]
        loss = jnp.sum(tot) * (-1.0 / B)
        acc_v[...] = jnp.full((L,), loss, jnp.float32)
        pltpu.sync_copy(acc_v, out_hbm)


_sc_loss = functools.partial(
    pl.kernel,
    mesh=plsc.VectorSubcoreMesh(core_axis_name="c", subcore_axis_name="s"),
    out_type=jax.ShapeDtypeStruct((L,), jnp.float32),
    compiler_params=pltpu.CompilerParams(needs_layout_passes=False),
    scratch_types=[
        pltpu.VMEM((ROWS_PER_SUB * NUM_ACTIONS,), jnp.float32),
        pltpu.VMEM((ROWS_PER_SUB,), jnp.int32),
        pltpu.VMEM((ROWS_PER_SUB,), jnp.float32),
        pltpu.VMEM((L,), jnp.float32),
        pltpu.VMEM_SHARED((NS * L,), jnp.float32),
        pltpu.VMEM((NS * L,), jnp.float32),
        pltpu.SMEM((1,), jnp.int32),
        pltpu.SemaphoreType.DMA,
        pltpu.SemaphoreType.DMA,
        pltpu.SemaphoreType.DMA,
    ],
)(_sc_body)


def kernel(q_pred, true_action, discounted_reward):
    out = _sc_loss(
        q_pred.reshape(B * NUM_ACTIONS), true_action.reshape(B), discounted_reward
    )
    return out[0]
